# Initial kernel scaffold; baseline (speedup 1.0000x reference)
import jax
import jax.numpy as jnp
from jax.experimental import pallas as pl


def kernel(x, tables):
    raise NotImplementedError("write your pallas kernel here")


# speedup vs baseline: 5.3572x; 5.3572x over previous
import functools

import jax
import jax.numpy as jnp
from jax import lax
from jax.experimental import pallas as pl
from jax.experimental.pallas import tpu as pltpu
from jax.experimental.pallas import tpu_sc as plsc

VOCAB = 100000
D = 128
NT = 8
LANES = 16


@functools.partial(jax.jit, static_argnums=(2, 3, 4))
def _sc_sum_embed(x_flat, tables_flat, N, NC, NS):
    NW = NC * NS
    TPW = N // NW
    C = 64
    n_chunks = TPW // C
    IC = NT * C
    NG = IC // 128

    mesh = plsc.VectorSubcoreMesh(core_axis_name="c", subcore_axis_name="s")

    @functools.partial(
        pl.kernel,
        mesh=mesh,
        out_type=jax.ShapeDtypeStruct((N, D), jnp.float32),
        scratch_types=[
            pltpu.VMEM((IC,), jnp.int32),
            pltpu.VMEM((IC, D), jnp.float32),
            pltpu.VMEM((C, D), jnp.float32),
            pltpu.SemaphoreType.DMA,
        ],
    )
    def k(x_hbm, tables_hbm, out_hbm, idx_v, rows_v, acc_v, sem):
        wid = lax.axis_index("s") * NC + lax.axis_index("c")
        wbase = wid * TPW
        offs = (lax.iota(jnp.int32, LANES) & (NT - 1)) * VOCAB

        def chunk_body(g, carry):
            tok0 = wbase + g * C
            ioff = pl.multiple_of(tok0 * NT, 512)
            pltpu.sync_copy(x_hbm.at[pl.ds(ioff, IC)], idx_v)
            for j in range(IC // LANES):
                idx_v[pl.ds(j * LANES, LANES)] = (
                    idx_v[pl.ds(j * LANES, LANES)] + offs
                )
            copies = [
                pltpu.async_copy(
                    tables_hbm.at[idx_v.at[pl.ds(kk * 128, 128)]],
                    rows_v.at[pl.ds(kk * 128, 128)],
                    sem,
                )
                for kk in range(NG)
            ]
            for cp in copies:
                cp.wait()

            def tok_body(t, carry2):
                base = t * NT
                for d in range(D // LANES):
                    sl = pl.ds(d * LANES, LANES)
                    s = rows_v[base, sl]
                    for r in range(1, NT):
                        s = s + rows_v[base + r, sl]
                    acc_v[t, sl] = s
                return carry2

            lax.fori_loop(0, C, tok_body, 0, unroll=2)
            pltpu.sync_copy(acc_v, out_hbm.at[pl.ds(tok0, C)])
            return carry

        lax.fori_loop(0, n_chunks, chunk_body, 0)

    return k(x_flat, tables_flat)


def kernel(x, tables):
    B, L, _ = x.shape
    N = B * L
    info = plsc.get_sparse_core_info()
    x_flat = x.reshape(N * NT)
    tables_flat = tables.reshape(NT * VOCAB, D)
    out = _sc_sum_embed(x_flat, tables_flat, N, info.num_cores,
                        info.num_subcores)
    return out.reshape(B, L, D)


# speedup vs baseline: 6.9745x; 1.3019x over previous
import functools

import jax
import jax.numpy as jnp
from jax import lax
from jax.experimental import pallas as pl
from jax.experimental.pallas import tpu as pltpu
from jax.experimental.pallas import tpu_sc as plsc

VOCAB = 100000
D = 128
NT = 8
LANES = 16


@functools.partial(jax.jit, static_argnums=(2, 3, 4))
def _sc_sum_embed(x_flat, tables_flat, N, NC, NS):
    NW = NC * NS
    TPW = N // NW
    C = 32
    n_chunks = TPW // C
    NCH2 = n_chunks // 2
    IC = NT * C
    NG = IC // 128

    mesh = plsc.VectorSubcoreMesh(core_axis_name="c", subcore_axis_name="s")

    @functools.partial(
        pl.kernel,
        mesh=mesh,
        out_type=jax.ShapeDtypeStruct((N, D), jnp.float32),
        scratch_types=[
            pltpu.VMEM((2, IC), jnp.int32),
            pltpu.VMEM((IC, D), jnp.float32),
            pltpu.VMEM((IC, D), jnp.float32),
            pltpu.VMEM((C, D), jnp.float32),
            pltpu.VMEM((C, D), jnp.float32),
            pltpu.SemaphoreType.DMA,
            pltpu.SemaphoreType.DMA,
        ],
    )
    def k(x_hbm, tables_hbm, out_hbm, idx_v, rows0_v, rows1_v, acc0_v,
          acc1_v, sem0, sem1):
        wid = lax.axis_index("s") * NC + lax.axis_index("c")
        wbase = wid * TPW
        rows_v = (rows0_v, rows1_v)
        acc_v = (acc0_v, acc1_v)
        sem = (sem0, sem1)
        offs = (lax.iota(jnp.int32, LANES) & (NT - 1)) * VOCAB

        def prep(gidx, b):
            ioff = pl.multiple_of((wbase + gidx * C) * NT, IC)
            pltpu.sync_copy(x_hbm.at[pl.ds(ioff, IC)], idx_v.at[b])
            for j in range(IC // LANES):
                idx_v[b, pl.ds(j * LANES, LANES)] = (
                    idx_v[b, pl.ds(j * LANES, LANES)] + offs
                )
            for kk in range(NG):
                pltpu.async_copy(
                    tables_hbm.at[idx_v.at[b].at[pl.ds(kk * 128, 128)]],
                    rows_v[b].at[pl.ds(kk * 128, 128)],
                    sem[b],
                )

        def consume(gidx, b):
            for kk in range(NG):
                pltpu.make_async_copy(
                    tables_hbm.at[idx_v.at[b].at[pl.ds(kk * 128, 128)]],
                    rows_v[b].at[pl.ds(kk * 128, 128)],
                    sem[b],
                ).wait()

            def tok_body(t, carry):
                base = t * NT
                for d in range(D // LANES):
                    sl = pl.ds(d * LANES, LANES)
                    s = rows_v[b][base, sl]
                    for r in range(1, NT):
                        s = s + rows_v[b][base + r, sl]
                    acc_v[b][t, sl] = s
                return carry

            lax.fori_loop(0, C, tok_body, 0, unroll=2)
            pltpu.sync_copy(acc_v[b], out_hbm.at[pl.ds(wbase + gidx * C, C)])

        prep(0, 0)

        def pair_body(g2, carry):
            g = 2 * g2
            prep(g + 1, 1)
            consume(g, 0)

            @pl.when(g2 < NCH2 - 1)
            def _():
                prep(g + 2, 0)

            consume(g + 1, 1)
            return carry

        lax.fori_loop(0, NCH2, pair_body, 0)

    return k(x_flat, tables_flat)


def kernel(x, tables):
    B, L, _ = x.shape
    N = B * L
    info = plsc.get_sparse_core_info()
    x_flat = x.reshape(N * NT)
    tables_flat = tables.reshape(NT * VOCAB, D)
    out = _sc_sum_embed(x_flat, tables_flat, N, info.num_cores,
                        info.num_subcores)
    return out.reshape(B, L, D)


# speedup vs baseline: 11.5177x; 1.6514x over previous
import functools

import jax
import jax.numpy as jnp
from jax import lax
from jax.experimental import pallas as pl
from jax.experimental.pallas import tpu as pltpu
from jax.experimental.pallas import tpu_sc as plsc

VOCAB = 100000
D = 128
NT = 8
LANES = 16


@functools.partial(jax.jit, static_argnums=(2, 3, 4))
def _sc_sum_embed(x_flat, tables_flat, N, NC, NS):
    NW = NC * NS
    TPW = N // NW
    C = 32
    n_chunks = TPW // C
    NCH2 = n_chunks // 2
    IC = NT * C
    NG = IC // 128

    mesh = plsc.VectorSubcoreMesh(core_axis_name="c", subcore_axis_name="s")

    @functools.partial(
        pl.kernel,
        mesh=mesh,
        out_type=jax.ShapeDtypeStruct((N, D), jnp.float32),
        scratch_types=[
            pltpu.VMEM((2, IC), jnp.int32),
            pltpu.VMEM((IC, D), jnp.float32),
            pltpu.VMEM((IC, D), jnp.float32),
            pltpu.VMEM((C, D), jnp.float32),
            pltpu.VMEM((C, D), jnp.float32),
            pltpu.SemaphoreType.DMA,
            pltpu.SemaphoreType.DMA,
        ],
    )
    def k(x_hbm, tables_hbm, out_hbm, idx_v, rows0_v, rows1_v, acc0_v,
          acc1_v, sem0, sem1):
        wid = lax.axis_index("s") * NC + lax.axis_index("c")
        wbase = wid * TPW
        rows_v = (rows0_v, rows1_v)
        acc_v = (acc0_v, acc1_v)
        sem = (sem0, sem1)
        offs = (lax.iota(jnp.int32, LANES) & (NT - 1)) * VOCAB

        def prep(gidx, b):
            ioff = pl.multiple_of((wbase + gidx * C) * NT, IC)
            pltpu.sync_copy(x_hbm.at[pl.ds(ioff, IC)], idx_v.at[b])
            for j in range(IC // LANES):
                idx_v[b, pl.ds(j * LANES, LANES)] = (
                    idx_v[b, pl.ds(j * LANES, LANES)] + offs
                )
            for kk in range(NG):
                pltpu.async_copy(
                    tables_hbm.at[idx_v.at[b].at[pl.ds(kk * 128, 128)]],
                    rows_v[b].at[pl.ds(kk * 128, 128)],
                    sem[b],
                )

        def consume(gidx, b):
            for kk in range(NG):
                pltpu.make_async_copy(
                    tables_hbm.at[idx_v.at[b].at[pl.ds(kk * 128, 128)]],
                    rows_v[b].at[pl.ds(kk * 128, 128)],
                    sem[b],
                ).wait()

            @plsc.parallel_loop(0, C, unroll=2)
            def tok_body(t):
                base = t * NT
                for d in range(D // LANES):
                    sl = pl.ds(d * LANES, LANES)
                    s = rows_v[b][base, sl]
                    for r in range(1, NT):
                        s = s + rows_v[b][base + r, sl]
                    acc_v[b][t, sl] = s
            pltpu.sync_copy(acc_v[b], out_hbm.at[pl.ds(wbase + gidx * C, C)])

        prep(0, 0)

        def pair_body(g2, carry):
            g = 2 * g2
            prep(g + 1, 1)
            consume(g, 0)

            @pl.when(g2 < NCH2 - 1)
            def _():
                prep(g + 2, 0)

            consume(g + 1, 1)
            return carry

        lax.fori_loop(0, NCH2, pair_body, 0)

    return k(x_flat, tables_flat)


def kernel(x, tables):
    B, L, _ = x.shape
    N = B * L
    info = plsc.get_sparse_core_info()
    x_flat = x.reshape(N * NT)
    tables_flat = tables.reshape(NT * VOCAB, D)
    out = _sc_sum_embed(x_flat, tables_flat, N, info.num_cores,
                        info.num_subcores)
    return out.reshape(B, L, D)


# speedup vs baseline: 11.9561x; 1.0381x over previous
import functools

import jax
import jax.numpy as jnp
from jax import lax
from jax.experimental import pallas as pl
from jax.experimental.pallas import tpu as pltpu
from jax.experimental.pallas import tpu_sc as plsc

VOCAB = 100000
D = 128
NT = 8
LANES = 16


@functools.partial(jax.jit, static_argnums=(2, 3, 4))
def _sc_sum_embed(x_flat, tables_flat, N, NC, NS):
    NW = NC * NS
    TPW = N // NW
    C = 16
    n_chunks = TPW // C
    NQ = n_chunks // 4
    IC = NT * C

    mesh = plsc.VectorSubcoreMesh(core_axis_name="c", subcore_axis_name="s")

    @functools.partial(
        pl.kernel,
        mesh=mesh,
        out_type=jax.ShapeDtypeStruct((N, D), jnp.float32),
        scratch_types=[
            pltpu.VMEM((4, IC), jnp.int32),
            pltpu.VMEM((4, IC, D), jnp.float32),
            pltpu.VMEM((4, C, D), jnp.float32),
            [pltpu.SemaphoreType.DMA] * 4,
            [pltpu.SemaphoreType.DMA] * 4,
            [pltpu.SemaphoreType.DMA] * 4,
        ],
    )
    def k(x_hbm, tables_hbm, out_hbm, idx_v, rows_v, acc_v, sem_i, sem_g,
          sem_o):
        wid = lax.axis_index("s") * NC + lax.axis_index("c")
        wbase = wid * TPW
        offs = (lax.iota(jnp.int32, LANES) & (NT - 1)) * VOCAB

        def fire_idx(c, b):
            ioff = pl.multiple_of((wbase + c * C) * NT, IC)
            pltpu.async_copy(x_hbm.at[pl.ds(ioff, IC)], idx_v.at[b],
                             sem_i[b])

        def fire_gather(c, b):
            ioff = pl.multiple_of((wbase + c * C) * NT, IC)
            pltpu.make_async_copy(x_hbm.at[pl.ds(ioff, IC)], idx_v.at[b],
                                  sem_i[b]).wait()
            for j in range(IC // LANES):
                idx_v[b, pl.ds(j * LANES, LANES)] = (
                    idx_v[b, pl.ds(j * LANES, LANES)] + offs
                )
            pltpu.async_copy(tables_hbm.at[idx_v.at[b]], rows_v.at[b],
                             sem_g[b])

        def consume(c, b, drain_pred):
            pltpu.make_async_copy(tables_hbm.at[idx_v.at[b]], rows_v.at[b],
                                  sem_g[b]).wait()
            if drain_pred is not None:
                @pl.when(drain_pred)
                def _():
                    pltpu.make_async_copy(
                        acc_v.at[b],
                        out_hbm.at[pl.ds(wbase + (c - 4) * C, C)],
                        sem_o[b],
                    ).wait()

            @plsc.parallel_loop(0, C, unroll=2)
            def tok_body(t):
                base = t * NT
                for d in range(D // LANES):
                    sl = pl.ds(d * LANES, LANES)
                    s = rows_v[b, base, sl]
                    for r in range(1, NT):
                        s = s + rows_v[b, base + r, sl]
                    acc_v[b, t, sl] = s

            pltpu.async_copy(acc_v.at[b], out_hbm.at[pl.ds(wbase + c * C, C)],
                             sem_o[b])

        for c in range(4):
            fire_idx(c, c)
        fire_gather(0, 0)
        fire_gather(1, 1)

        def quad_body(q, carry):
            c0 = 4 * q
            for j in range(4):
                c = c0 + j
                b = j
                bg = (j + 2) & 3
                if j < 2:
                    fire_gather(c + 2, bg)
                else:
                    @pl.when(q < NQ - 1)
                    def _(c=c, bg=bg):
                        fire_gather(c + 2, bg)
                consume(c, b, drain_pred=q > 0)
                @pl.when(q < NQ - 1)
                def _(c=c, b=b):
                    fire_idx(c + 4, b)
            return carry

        lax.fori_loop(0, NQ, quad_body, 0)
        for j in range(4):
            c = n_chunks - 4 + j
            pltpu.make_async_copy(
                acc_v.at[j],
                out_hbm.at[pl.ds(wbase + c * C, C)],
                sem_o[j],
            ).wait()

    return k(x_flat, tables_flat)


def kernel(x, tables):
    B, L, _ = x.shape
    N = B * L
    info = plsc.get_sparse_core_info()
    x_flat = x.reshape(N * NT)
    tables_flat = tables.reshape(NT * VOCAB, D)
    out = _sc_sum_embed(x_flat, tables_flat, N, info.num_cores,
                        info.num_subcores)
    return out.reshape(B, L, D)


# speedup vs baseline: 13.2794x; 1.1107x over previous
import functools

import jax
import jax.numpy as jnp
from jax import lax
from jax.experimental import pallas as pl
from jax.experimental.pallas import tpu as pltpu
from jax.experimental.pallas import tpu_sc as plsc

VOCAB = 100000
D = 128
NT = 8
LANES = 16


@functools.partial(jax.jit, static_argnums=(2, 3, 4))
def _sc_sum_embed(x_flat, tables_flat, N, NC, NS):
    NW = NC * NS
    TPW = N // NW
    C = 16
    n_chunks = TPW // C
    NQ = n_chunks // 4
    IC = NT * C

    mesh = plsc.VectorSubcoreMesh(core_axis_name="c", subcore_axis_name="s")

    @functools.partial(
        pl.kernel,
        mesh=mesh,
        out_type=jax.ShapeDtypeStruct((N, D), jnp.float32),
        scratch_types=[
            pltpu.VMEM((8, IC), jnp.int32),
            pltpu.VMEM((4, IC, D), jnp.float32),
            pltpu.VMEM((4, C, D), jnp.float32),
            [pltpu.SemaphoreType.DMA] * 8,
            [pltpu.SemaphoreType.DMA] * 4,
            [pltpu.SemaphoreType.DMA] * 4,
        ],
    )
    def k(x_hbm, tables_hbm, out_hbm, idx_v, rows_v, acc_v, sem_i, sem_g,
          sem_o):
        wid = lax.axis_index("s") * NC + lax.axis_index("c")
        wbase = wid * TPW
        offs = (lax.iota(jnp.int32, LANES) & (NT - 1)) * VOCAB

        def fire_idx(c, bi):
            ioff = pl.multiple_of((wbase + c * C) * NT, IC)
            pltpu.async_copy(x_hbm.at[pl.ds(ioff, IC)], idx_v.at[bi],
                             sem_i[bi])

        def fire_gather(c, b, bi):
            ioff = pl.multiple_of((wbase + c * C) * NT, IC)
            pltpu.make_async_copy(x_hbm.at[pl.ds(ioff, IC)], idx_v.at[bi],
                                  sem_i[bi]).wait()
            for j in range(IC // LANES):
                idx_v[bi, pl.ds(j * LANES, LANES)] = (
                    idx_v[bi, pl.ds(j * LANES, LANES)] + offs
                )
            pltpu.async_copy(tables_hbm.at[idx_v.at[bi]], rows_v.at[b],
                             sem_g[b])

        def consume(c, b, bi, drain_pred):
            pltpu.make_async_copy(tables_hbm.at[idx_v.at[bi]], rows_v.at[b],
                                  sem_g[b]).wait()
            if drain_pred is not None:
                @pl.when(drain_pred)
                def _():
                    pltpu.make_async_copy(
                        acc_v.at[b],
                        out_hbm.at[pl.ds(wbase + (c - 4) * C, C)],
                        sem_o[b],
                    ).wait()
            else:
                pltpu.make_async_copy(
                    acc_v.at[b],
                    out_hbm.at[pl.ds(wbase + (c - 4) * C, C)],
                    sem_o[b],
                ).wait()

            @plsc.parallel_loop(0, C, unroll=4)
            def tok_body(t):
                base = t * NT
                for d in range(D // LANES):
                    sl = pl.ds(d * LANES, LANES)
                    s = rows_v[b, base, sl]
                    for r in range(1, NT):
                        s = s + rows_v[b, base + r, sl]
                    acc_v[b, t, sl] = s

            pltpu.async_copy(acc_v.at[b], out_hbm.at[pl.ds(wbase + c * C, C)],
                             sem_o[b])

        for c in range(8):
            fire_idx(c, c)
        for c in range(3):
            fire_gather(c, c, c)

        NO = n_chunks // 8

        def octo_body(o, carry):
            c0 = 8 * o
            for j in range(8):
                c = c0 + j
                b = j & 3
                bi = j
                bg = (j + 3) & 3
                big = (j + 3) & 7
                if j < 5:
                    fire_gather(c + 3, bg, big)
                else:
                    @pl.when(o < NO - 1)
                    def _(c=c, bg=bg, big=big):
                        fire_gather(c + 3, bg, big)
                consume(c, b, bi, drain_pred=(o > 0) if j < 4 else None)
                @pl.when(o < NO - 1)
                def _(c=c, bi=bi):
                    fire_idx(c + 8, bi)
            return carry

        lax.fori_loop(0, NO, octo_body, 0)
        for j in range(4):
            c = n_chunks - 4 + j
            pltpu.make_async_copy(
                acc_v.at[j],
                out_hbm.at[pl.ds(wbase + c * C, C)],
                sem_o[j],
            ).wait()

    return k(x_flat, tables_flat)


def kernel(x, tables):
    B, L, _ = x.shape
    N = B * L
    info = plsc.get_sparse_core_info()
    x_flat = x.reshape(N * NT)
    tables_flat = tables.reshape(NT * VOCAB, D)
    out = _sc_sum_embed(x_flat, tables_flat, N, info.num_cores,
                        info.num_subcores)
    return out.reshape(B, L, D)
